# baseline (speedup 1.0000x reference)
import functools
import math

import jax
import jax.numpy as jnp
from jax import lax
from jax.experimental import pallas as pl
from jax.experimental.pallas import tpu as pltpu
from jax.experimental.pallas import tpu_sc as plsc

EMB = 64
SCALE = math.sqrt(EMB)
LANES = 16
G = 128
K = 4
CHUNK = G * K


@functools.cache
def _build(B, V):
    info = plsc.get_sparse_core_info()
    NC, NS = info.num_cores, info.num_subcores
    NW = NC * NS
    assert B % (NW * CHUNK) == 0, (B, NW, CHUNK)
    b_per_w = B // NW
    n_chunks = b_per_w // CHUNK

    mesh = plsc.VectorSubcoreMesh(core_axis_name="c", subcore_axis_name="s")

    @functools.partial(
        pl.kernel,
        mesh=mesh,
        out_type=jax.ShapeDtypeStruct((B, EMB), jnp.float32),
        scratch_types=[
            pltpu.VMEM((K, G), jnp.int32),
            pltpu.VMEM((CHUNK, EMB), jnp.float32),
            pltpu.SemaphoreType.DMA,
        ],
        compiler_params=pltpu.CompilerParams(use_tc_tiling_on_sc=False),
    )
    def emb_kernel(tokens_hbm, table_hbm, out_hbm, idx_v, rows_v, sem):
        wid = lax.axis_index("s") * NC + lax.axis_index("c")
        base_row = wid * (b_per_w // G)

        def chunk_body(g, carry):
            row0 = base_row + g * K
            pltpu.sync_copy(tokens_hbm.at[pl.ds(row0, K)], idx_v)
            copies = [
                pltpu.async_copy(
                    table_hbm.at[idx_v.at[j]],
                    rows_v.at[pl.ds(j * G, G)],
                    sem,
                )
                for j in range(K)
            ]
            for c in copies:
                c.wait()

            def scale_body(r, carry2):
                for c in range(EMB // LANES):
                    sl = pl.ds(c * LANES, LANES)
                    rows_v[r, sl] = rows_v[r, sl] * SCALE
                return carry2

            lax.fori_loop(0, CHUNK, scale_body, 0)
            pltpu.sync_copy(rows_v, out_hbm.at[pl.ds(row0 * G, CHUNK)])
            return carry

        lax.fori_loop(0, n_chunks, chunk_body, 0)

    return emb_kernel


def kernel(tokens, table):
    B = tokens.size
    V = table.shape[0]
    tokens2d = tokens.reshape(B // G, G).astype(jnp.int32)
    out = _build(B, V)(tokens2d, table)
    return out.reshape(*tokens.shape, EMB)


# speedup vs baseline: 1.1345x; 1.1345x over previous
import functools
import math

import jax
import jax.numpy as jnp
from jax import lax
from jax.experimental import pallas as pl
from jax.experimental.pallas import tpu as pltpu
from jax.experimental.pallas import tpu_sc as plsc

EMB = 64
SCALE = math.sqrt(EMB)
LANES = 16
G = 128
K = 4
CHUNK = G * K
NBUF = 3


@functools.cache
def _build(B, V):
    info = plsc.get_sparse_core_info()
    NC, NS = info.num_cores, info.num_subcores
    NW = NC * NS
    assert B % (NW * CHUNK) == 0, (B, NW, CHUNK)
    b_per_w = B // NW
    n_chunks = b_per_w // CHUNK
    idx_rows = b_per_w // G
    assert n_chunks >= 2 * NBUF and n_chunks % NBUF == 2, n_chunks

    mesh = plsc.VectorSubcoreMesh(core_axis_name="c", subcore_axis_name="s")

    @functools.partial(
        pl.kernel,
        mesh=mesh,
        out_type=jax.ShapeDtypeStruct((B, EMB), jnp.float32),
        scratch_types=[
            pltpu.VMEM((idx_rows, G), jnp.int32),
            pltpu.VMEM((NBUF, CHUNK, EMB), jnp.float32),
            pltpu.SemaphoreType.DMA((NBUF,)),
            pltpu.SemaphoreType.DMA((NBUF,)),
        ],
        compiler_params=pltpu.CompilerParams(use_tc_tiling_on_sc=False),
    )
    def emb_kernel(tokens_hbm, table_hbm, out_hbm, idx_all, rows_v, gsem, ssem):
        wid = lax.axis_index("s") * NC + lax.axis_index("c")
        out_base = wid * b_per_w

        pltpu.sync_copy(tokens_hbm.at[pl.ds(wid * idx_rows, idx_rows)], idx_all)

        def fire(g, b):
            for j in range(K):
                pltpu.async_copy(
                    table_hbm.at[idx_all.at[g * K + j]],
                    rows_v.at[b, pl.ds(j * G, G)],
                    gsem.at[b],
                )

        def wait_gather(b):
            pltpu.make_async_copy(
                out_hbm.at[pl.ds(0, CHUNK)], rows_v.at[b], gsem.at[b]
            ).wait()

        def store(g, b):
            pltpu.async_copy(
                rows_v.at[b],
                out_hbm.at[pl.ds(out_base + g * CHUNK, CHUNK)],
                ssem.at[b],
            )

        def wait_store(b):
            pltpu.make_async_copy(
                out_hbm.at[pl.ds(0, CHUNK)], rows_v.at[b], ssem.at[b]
            ).wait()

        def scale(b):
            rows_b = rows_v.at[b]

            @plsc.parallel_loop(0, CHUNK, step=1, unroll=8)
            def _(r):
                for c in range(EMB // LANES):
                    sl = pl.ds(c * LANES, LANES)
                    rows_b[r, sl] = rows_b[r, sl] * SCALE

        def step(g, b, do_wait_store, do_fire):
            b2 = (b + 2) % NBUF
            wait_gather(b)
            scale(b)
            store(g, b)
            if do_fire:
                if do_wait_store:
                    wait_store(b2)
                fire(g + 2, b2)

        fire(0, 0)
        fire(1, 1)
        step(0, 0, False, True)
        step(1, 1, True, True)
        step(2, 2, True, True)

        def body(p, carry):
            g0 = NBUF * p
            for b in range(NBUF):
                step(g0 + b, b, True, True)
            return carry

        lax.fori_loop(1, n_chunks // NBUF, body, 0)

        step(n_chunks - 2, 0, False, False)
        step(n_chunks - 1, 1, False, False)
        wait_store(2)
        wait_store(0)
        wait_store(1)

    return emb_kernel


def kernel(tokens, table):
    B = tokens.size
    V = table.shape[0]
    tokens2d = tokens.reshape(B // G, G).astype(jnp.int32)
    out = _build(B, V)(tokens2d, table)
    return out.reshape(*tokens.shape, EMB)
